# baseline (speedup 1.0000x reference)
import functools

import jax
import jax.numpy as jnp
from jax import lax
from jax.experimental import pallas as pl
from jax.experimental.pallas import tpu as pltpu
from jax.experimental.pallas import tpu_sc as plsc

_BATCH = 16384
_HIST = 50
_HPAD = 56
_DIM = 64
_DPAD = 128
_NC = 2
_NS = 16
_NW = _NC * _NS
_B_PER_W = _BATCH // _NW
_CB = 4
_M = _B_PER_W // _CB
_NBUF = 2


def _embed_body(weight_hbm, idx_hbm, out_hbm, idx_v, rows_v,
                gsem0, gsem1, ssem0, ssem1):
    wid = lax.axis_index("s") * _NC + lax.axis_index("c")
    base_b = wid * _B_PER_W
    pltpu.sync_copy(idx_hbm.at[wid], idx_v)
    gsems = (gsem0, gsem1)
    ssems = (ssem0, ssem1)

    def _start_fill(g, b):
        for bi in range(_CB):
            pltpu.async_copy(
                weight_hbm.at[idx_v.at[g * _CB + bi]],
                rows_v.at[b, bi],
                gsems[b],
            )

    def _drain_fill(b):
        pltpu.make_async_copy(
            out_hbm.at[pl.ds(0, _CB)],
            rows_v.at[b],
            gsems[b],
        ).wait()

    def _out_slice(g):
        return out_hbm.at[pl.ds(base_b + g * _CB, _CB)]

    def _start_store(g, b):
        pltpu.async_copy(rows_v.at[b], _out_slice(g), ssems[b])

    def _drain_store(g, b):
        pltpu.make_async_copy(rows_v.at[b], _out_slice(g), ssems[b]).wait()

    _start_fill(0, 0)

    def body(i, carry):
        for b in range(_NBUF):
            g = i * _NBUF + b
            nxt = g + 1
            nb = (b + 1) % _NBUF

            @pl.when(nxt < _M)
            def _():
                @pl.when(nxt >= _NBUF)
                def _():
                    _drain_store(nxt - _NBUF, nb)

                _start_fill(nxt, nb)

            _drain_fill(b)
            _start_store(g, b)
        return carry

    lax.fori_loop(0, _M // _NBUF, body, 0)
    _drain_store(_M - 2, (_M - 2) % _NBUF)
    _drain_store(_M - 1, (_M - 1) % _NBUF)


@functools.partial(
    pl.kernel,
    out_type=jax.ShapeDtypeStruct((_BATCH, _HPAD, _DIM), jnp.float32),
    mesh=plsc.VectorSubcoreMesh(core_axis_name="c", subcore_axis_name="s"),
    compiler_params=pltpu.CompilerParams(use_tc_tiling_on_sc=False),
    scratch_types=[
        pltpu.VMEM((_B_PER_W, _HPAD), jnp.int32),
        pltpu.VMEM((_NBUF, _CB, _HPAD, _DIM), jnp.float32),
        pltpu.SemaphoreType.DMA,
        pltpu.SemaphoreType.DMA,
        pltpu.SemaphoreType.DMA,
        pltpu.SemaphoreType.DMA,
    ],
)
def _embed_kernel(weight_hbm, idx_hbm, out_hbm, idx_v, rows_v,
                  gsem0, gsem1, ssem0, ssem1):
    _embed_body(weight_hbm, idx_hbm, out_hbm, idx_v, rows_v,
                gsem0, gsem1, ssem0, ssem1)


def kernel(input_, weight):
    idx = input_.reshape(_NW, _B_PER_W, _HIST)
    idx = jnp.pad(idx, ((0, 0), (0, 0), (0, _HPAD - _HIST)))
    out = _embed_kernel(weight, idx)
    return out[:, :_HIST, :]


# speedup vs baseline: 2.7713x; 2.7713x over previous
import functools

import jax
import jax.numpy as jnp
from jax import lax
from jax.experimental import pallas as pl
from jax.experimental.pallas import tpu as pltpu
from jax.experimental.pallas import tpu_sc as plsc

_BATCH = 16384
_HIST = 50
_DIM = 64
_NC = 2
_NS = 16
_NW = _NC * _NS
_B_TOTAL = _BATCH * _HIST
_B_PER_W = _B_TOTAL // _NW
_CHUNK = 128
_SUB = 4
_ROWS = _CHUNK * _SUB
_M = _B_PER_W // _ROWS
_NBUF = 2


def _embed_body(weight_hbm, idx_hbm, out_hbm, idx_v, rows_v,
                gsem0, gsem1, ssem0, ssem1):
    wid = lax.axis_index("s") * _NC + lax.axis_index("c")
    pltpu.sync_copy(idx_hbm.at[wid], idx_v)
    gsems = (gsem0, gsem1)
    ssems = (ssem0, ssem1)

    def _start_fill(g, b):
        for j in range(_SUB):
            pltpu.async_copy(
                weight_hbm.at[idx_v.at[g, j]],
                rows_v.at[b, pl.ds(j * _CHUNK, _CHUNK)],
                gsems[b],
            )

    def _drain_fill(b):
        pltpu.make_async_copy(
            weight_hbm.at[pl.ds(0, _ROWS)], rows_v.at[b], gsems[b]
        ).wait()

    def _start_store(g, b):
        pltpu.async_copy(rows_v.at[b], out_hbm.at[wid, g], ssems[b])

    def _drain_store(g, b):
        pltpu.make_async_copy(
            rows_v.at[b], out_hbm.at[wid, g], ssems[b]
        ).wait()

    _start_fill(0, 0)

    def body(i, carry):
        for b in range(_NBUF):
            g = i * _NBUF + b
            nxt = g + 1
            nb = (b + 1) % _NBUF

            @pl.when(nxt < _M)
            def _():
                @pl.when(nxt >= _NBUF)
                def _():
                    _drain_store(nxt - _NBUF, nb)

                _start_fill(nxt, nb)

            _drain_fill(b)
            _start_store(g, b)
        return carry

    lax.fori_loop(0, _M // _NBUF, body, 0)
    _drain_store(_M - 2, (_M - 2) % _NBUF)
    _drain_store(_M - 1, (_M - 1) % _NBUF)


@functools.partial(
    pl.kernel,
    out_type=jax.ShapeDtypeStruct((_NW, _M, _ROWS, _DIM), jnp.float32),
    mesh=plsc.VectorSubcoreMesh(core_axis_name="c", subcore_axis_name="s"),
    compiler_params=pltpu.CompilerParams(use_tc_tiling_on_sc=False),
    scratch_types=[
        pltpu.VMEM((_M, _SUB, _CHUNK), jnp.int32),
        pltpu.VMEM((_NBUF, _ROWS, _DIM), jnp.float32),
        pltpu.SemaphoreType.DMA,
        pltpu.SemaphoreType.DMA,
        pltpu.SemaphoreType.DMA,
        pltpu.SemaphoreType.DMA,
    ],
)
def _embed_kernel(weight_hbm, idx_hbm, out_hbm, idx_v, rows_v,
                  gsem0, gsem1, ssem0, ssem1):
    _embed_body(weight_hbm, idx_hbm, out_hbm, idx_v, rows_v,
                gsem0, gsem1, ssem0, ssem1)


def kernel(input_, weight):
    idx = input_.reshape(_NW, _M, _SUB, _CHUNK)
    out = _embed_kernel(weight, idx)
    return out.reshape(_BATCH, _HIST, _DIM)
